# baseline (speedup 1.0000x reference)
import dataclasses

import jax
import jax.numpy as jnp
from jax import lax
from jax.experimental import pallas as pl
from jax.experimental.pallas import tpu as pltpu
from jax.experimental.pallas import tpu_sc as plsc

NBX = 512
NBY = 512
NBINS = NBX * NBY
BSX = 1.0 / NBX
BSY = 1.0 / NBY
INV_BSX = float(NBX)
INV_BSY = float(NBY)
BIN_AREA = BSX * BSY
H2 = BSX * BSY
N_ITERS = 40

N_NODES = 1_000_000
NW = 32
CHUNK = 2048
NCHUNK = 16
ROWS = CHUNK // 128
NGRID = 4
STRIPE = NBINS // 16


def _splat_body(pos_hbm, sx_hbm, sy_hbm, out_hbm,
                pxb, pyb, sxb, syb,
                i00b, i10b, i01b, i11b,
                v00b, v10b, v01b, v11b,
                tmp, grid_sh, sem):
    cid = lax.axis_index("c")
    sid = lax.axis_index("s")
    goff = (sid // 4) * NBINS
    iota16 = lax.iota(jnp.int32, 16)

    @pl.loop(0, STRIPE // 16)
    def _(i):
        tmp[pl.ds(i * 16, 16)] = jnp.zeros((16,), jnp.float32)

    for g in range(NGRID):
        pltpu.sync_copy(tmp, grid_sh.at[pl.ds(g * NBINS + sid * STRIPE,
                                              STRIPE)])
    plsc.subcore_barrier()

    @pl.loop(0, NCHUNK)
    def _(t):
        wid = cid * 16 + sid
        raw = (t * NW + wid) * CHUNK
        base = jnp.minimum(raw, N_NODES - CHUNK)
        pltpu.sync_copy(pos_hbm.at[pl.ds(base, CHUNK)], pxb)
        pltpu.sync_copy(pos_hbm.at[pl.ds(N_NODES + base, CHUNK)], pyb)
        pltpu.sync_copy(sx_hbm.at[pl.ds(base, CHUNK)], sxb)
        pltpu.sync_copy(sy_hbm.at[pl.ds(base, CHUNK)], syb)

        @pl.loop(0, ROWS)
        def _(r):
            def corners(sl, valid):
                gx = pxb[sl] * INV_BSX - 0.5
                gy = pyb[sl] * INV_BSY - 0.5
                ix0 = gx.astype(jnp.int32)
                iy0 = gy.astype(jnp.int32)
                wx = jnp.clip(gx - ix0.astype(jnp.float32), 0.0, 1.0)
                wy = jnp.clip(gy - iy0.astype(jnp.float32), 0.0, 1.0)
                area = jnp.where(valid, sxb[sl] * syb[sl], 0.0)
                iy0g = jnp.minimum(iy0, NBY - 1) + goff
                iy1g = jnp.minimum(iy0 + 1, NBY - 1) + goff
                bx0 = ix0 * NBY
                bx1 = jnp.minimum(ix0 + 1, NBX - 1) * NBY
                ax1 = wx * area
                ax0 = area - ax1
                return ((bx0 + iy0g, bx1 + iy0g, bx0 + iy1g, bx1 + iy1g),
                        (ax0 - ax0 * wy, ax1 - ax1 * wy, ax0 * wy, ax1 * wy))

            for c in range(8):
                off = r * 128 + c * 16
                sl = pl.ds(c * 16, 16)
                idxs, vals = corners(pl.ds(off, 16),
                                     base + off + iota16 >= raw)
                for ibuf, x in zip((i00b, i10b, i01b, i11b), idxs):
                    ibuf[sl] = x
                for vbuf, x in zip((v00b, v10b, v01b, v11b), vals):
                    vbuf[sl] = x
            pltpu.sync_copy(v00b, grid_sh.at[i00b], add=True)
            pltpu.sync_copy(v10b, grid_sh.at[i10b], add=True)
            pltpu.sync_copy(v01b, grid_sh.at[i01b], add=True)
            pltpu.sync_copy(v11b, grid_sh.at[i11b], add=True)

    plsc.subcore_barrier()

    for g in range(NGRID):
        src = pl.ds(g * NBINS + sid * STRIPE, STRIPE)
        dst = pl.ds((cid * NGRID + g) * NBINS + sid * STRIPE, STRIPE)
        pltpu.sync_copy(grid_sh.at[src], tmp)
        pltpu.sync_copy(tmp, out_hbm.at[dst])


def _splat(pos, sx, sy):
    mesh = plsc.VectorSubcoreMesh(core_axis_name="c", subcore_axis_name="s")
    cp = pltpu.CompilerParams()
    if "needs_layout_passes" in pltpu.CompilerParams.__dataclass_fields__:
        cp = dataclasses.replace(cp, needs_layout_passes=False)
    k = pl.kernel(
        _splat_body,
        compiler_params=cp,
        out_type=jax.ShapeDtypeStruct((2 * NGRID * NBINS,), jnp.float32),
        mesh=mesh,
        scratch_types=(
            [pltpu.VMEM((CHUNK,), jnp.float32)] * 4
            + [pltpu.VMEM((128,), jnp.int32)] * 4
            + [pltpu.VMEM((128,), jnp.float32)] * 4
            + [
                pltpu.VMEM((STRIPE,), jnp.float32),
                pltpu.VMEM_SHARED((NGRID * NBINS,), jnp.float32),
                pltpu.SemaphoreType.DMA,
            ]
        ),
    )
    return k(pos, sx, sy)


def _dense_body(p_ref, out_ref):
    acc = p_ref[0].astype(jnp.float32)
    for g in range(1, 2 * NGRID):
        acc = acc + p_ref[g].astype(jnp.float32)
    rho = acc * (1.0 / BIN_AREA)
    rhs = rho - jnp.mean(rho)

    def step(_, phi):
        up = jnp.concatenate([phi[:1, :], phi[:-1, :]], axis=0)
        down = jnp.concatenate([phi[1:, :], phi[-1:, :]], axis=0)
        left = jnp.concatenate([phi[:, :1], phi[:, :-1]], axis=1)
        right = jnp.concatenate([phi[:, 1:], phi[:, -1:]], axis=1)
        return 0.25 * (up + down + left + right - H2 * rhs)

    phi = lax.fori_loop(0, N_ITERS, step, jnp.zeros((NBX, NBY), jnp.float32))
    vx = jnp.concatenate([
        -(phi[1:2, :] - phi[0:1, :]) * INV_BSX,
        -(phi[2:, :] - phi[:-2, :]) * (0.5 * INV_BSX),
        -(phi[-1:, :] - phi[-2:-1, :]) * INV_BSX,
    ], axis=0)
    vy = jnp.concatenate([
        -(phi[:, 1:2] - phi[:, 0:1]) * INV_BSY,
        -(phi[:, 2:] - phi[:, :-2]) * (0.5 * INV_BSY),
        -(phi[:, -1:] - phi[:, -2:-1]) * INV_BSY,
    ], axis=1)
    energy = 0.5 * jnp.sum(rho * (vx * vx + vy * vy)) * BIN_AREA
    out_ref[...] = jnp.broadcast_to(energy, (1, 1))


def _dense(partials):
    return pl.pallas_call(
        _dense_body,
        out_shape=jax.ShapeDtypeStruct((1, 1), jnp.float32),
    )(partials)


def kernel(pos, node_size_x, node_size_y):
    flat = _splat(pos, node_size_x, node_size_y)
    partials = flat.reshape(2 * NGRID, NBX, NBY)
    energy = _dense(partials)
    return energy.reshape(1)


# speedup vs baseline: 1.0025x; 1.0025x over previous
import dataclasses

import jax
import jax.numpy as jnp
from jax import lax
from jax.experimental import pallas as pl
from jax.experimental.pallas import tpu as pltpu
from jax.experimental.pallas import tpu_sc as plsc

NBX = 512
NBY = 512
NBINS = NBX * NBY
BSX = 1.0 / NBX
BSY = 1.0 / NBY
INV_BSX = float(NBX)
INV_BSY = float(NBY)
BIN_AREA = BSX * BSY
H2 = BSX * BSY
N_ITERS = 40

N_NODES = 1_000_000
NW = 32
CHUNK = 2048
NCHUNK = 16
ROWS = CHUNK // 128
NGRID = 4
STRIPE = NBINS // 16


def _splat_body(pos_hbm, sx_hbm, sy_hbm, out_hbm,
                pxb, pyb, sxb, syb,
                i00b, i10b, i01b, i11b,
                v00b, v10b, v01b, v11b,
                tmp, grid_sh, sem):
    cid = lax.axis_index("c")
    sid = lax.axis_index("s")
    goff = (sid // 4) * NBINS
    iota16 = lax.iota(jnp.int32, 16)

    @pl.loop(0, STRIPE // 16)
    def _(i):
        tmp[pl.ds(i * 16, 16)] = jnp.zeros((16,), jnp.float32)

    for g in range(NGRID):
        pltpu.sync_copy(tmp, grid_sh.at[pl.ds(g * NBINS + sid * STRIPE,
                                              STRIPE)])
    plsc.subcore_barrier()

    @pl.loop(0, NCHUNK)
    def _(t):
        wid = cid * 16 + sid
        raw = (t * NW + wid) * CHUNK
        base = jnp.minimum(raw, N_NODES - CHUNK)
        pltpu.sync_copy(pos_hbm.at[pl.ds(base, CHUNK)], pxb)
        pltpu.sync_copy(pos_hbm.at[pl.ds(N_NODES + base, CHUNK)], pyb)
        pltpu.sync_copy(sx_hbm.at[pl.ds(base, CHUNK)], sxb)
        pltpu.sync_copy(sy_hbm.at[pl.ds(base, CHUNK)], syb)

        @pl.loop(0, ROWS)
        def _(r):
            def corners(sl, valid):
                gx = pxb[sl] * INV_BSX - 0.5
                gy = pyb[sl] * INV_BSY - 0.5
                ix0 = gx.astype(jnp.int32)
                iy0 = gy.astype(jnp.int32)
                wx = jnp.clip(gx - ix0.astype(jnp.float32), 0.0, 1.0)
                wy = jnp.clip(gy - iy0.astype(jnp.float32), 0.0, 1.0)
                area = jnp.where(valid, sxb[sl] * syb[sl], 0.0)
                iy0g = jnp.minimum(iy0, NBY - 1) + goff
                iy1g = jnp.minimum(iy0 + 1, NBY - 1) + goff
                bx0 = ix0 * NBY
                bx1 = jnp.minimum(ix0 + 1, NBX - 1) * NBY
                ax1 = wx * area
                ax0 = area - ax1
                return ((bx0 + iy0g, bx1 + iy0g, bx0 + iy1g, bx1 + iy1g),
                        (ax0 - ax0 * wy, ax1 - ax1 * wy, ax0 * wy, ax1 * wy))

            for c in range(8):
                off = r * 128 + c * 16
                sl = pl.ds(c * 16, 16)
                idxs, vals = corners(pl.ds(off, 16),
                                     base + off + iota16 >= raw)
                for ibuf, x in zip((i00b, i10b, i01b, i11b), idxs):
                    ibuf[sl] = x
                for vbuf, x in zip((v00b, v10b, v01b, v11b), vals):
                    vbuf[sl] = x
            pltpu.sync_copy(v00b, grid_sh.at[i00b], add=True)
            pltpu.sync_copy(v10b, grid_sh.at[i10b], add=True)
            pltpu.sync_copy(v01b, grid_sh.at[i01b], add=True)
            pltpu.sync_copy(v11b, grid_sh.at[i11b], add=True)

    plsc.subcore_barrier()

    for g in range(NGRID):
        src = pl.ds(g * NBINS + sid * STRIPE, STRIPE)
        dst = pl.ds((cid * NGRID + g) * NBINS + sid * STRIPE, STRIPE)
        pltpu.sync_copy(grid_sh.at[src], tmp)
        pltpu.sync_copy(tmp, out_hbm.at[dst])


def _splat(pos, sx, sy):
    mesh = plsc.VectorSubcoreMesh(core_axis_name="c", subcore_axis_name="s")
    cp = pltpu.CompilerParams()
    if "needs_layout_passes" in pltpu.CompilerParams.__dataclass_fields__:
        cp = dataclasses.replace(cp, needs_layout_passes=False)
    k = pl.kernel(
        _splat_body,
        compiler_params=cp,
        out_type=jax.ShapeDtypeStruct((2 * NGRID * NBINS,), jnp.float32),
        mesh=mesh,
        scratch_types=(
            [pltpu.VMEM((CHUNK,), jnp.float32)] * 4
            + [pltpu.VMEM((128,), jnp.int32)] * 4
            + [pltpu.VMEM((128,), jnp.float32)] * 4
            + [
                pltpu.VMEM((STRIPE,), jnp.float32),
                pltpu.VMEM_SHARED((NGRID * NBINS,), jnp.float32),
                pltpu.SemaphoreType.DMA,
            ]
        ),
    )
    return k(pos, sx, sy)


def _dense_body(p_ref, out_ref):
    acc = p_ref[0].astype(jnp.float32)
    for g in range(1, 2 * NGRID):
        acc = acc + p_ref[g].astype(jnp.float32)
    rho = acc * (1.0 / BIN_AREA)
    rhs = rho - jnp.mean(rho)
    src = (0.25 * H2) * rhs

    def step(_, phi):
        up = jnp.concatenate([phi[:1, :], phi[:-1, :]], axis=0)
        down = jnp.concatenate([phi[1:, :], phi[-1:, :]], axis=0)
        left = jnp.concatenate([phi[:, :1], phi[:, :-1]], axis=1)
        right = jnp.concatenate([phi[:, 1:], phi[:, -1:]], axis=1)
        return 0.25 * (up + down + left + right) - src

    phi = lax.fori_loop(0, N_ITERS, step, jnp.zeros((NBX, NBY), jnp.float32))
    vx = jnp.concatenate([
        -(phi[1:2, :] - phi[0:1, :]) * INV_BSX,
        -(phi[2:, :] - phi[:-2, :]) * (0.5 * INV_BSX),
        -(phi[-1:, :] - phi[-2:-1, :]) * INV_BSX,
    ], axis=0)
    vy = jnp.concatenate([
        -(phi[:, 1:2] - phi[:, 0:1]) * INV_BSY,
        -(phi[:, 2:] - phi[:, :-2]) * (0.5 * INV_BSY),
        -(phi[:, -1:] - phi[:, -2:-1]) * INV_BSY,
    ], axis=1)
    energy = 0.5 * jnp.sum(rho * (vx * vx + vy * vy)) * BIN_AREA
    out_ref[...] = jnp.broadcast_to(energy, (1, 1))


def _dense(partials):
    return pl.pallas_call(
        _dense_body,
        out_shape=jax.ShapeDtypeStruct((1, 1), jnp.float32),
    )(partials)


def kernel(pos, node_size_x, node_size_y):
    flat = _splat(pos, node_size_x, node_size_y)
    partials = flat.reshape(2 * NGRID, NBX, NBY)
    energy = _dense(partials)
    return energy.reshape(1)


# speedup vs baseline: 1.1863x; 1.1833x over previous
import dataclasses

import jax
import jax.numpy as jnp
from jax import lax
from jax.experimental import pallas as pl
from jax.experimental.pallas import tpu as pltpu
from jax.experimental.pallas import tpu_sc as plsc

NBX = 512
NBY = 512
NBINS = NBX * NBY
BSX = 1.0 / NBX
BSY = 1.0 / NBY
INV_BSX = float(NBX)
INV_BSY = float(NBY)
BIN_AREA = BSX * BSY
H2 = BSX * BSY
N_ITERS = 40

N_NODES = 1_000_000
NW = 32
CHUNK = 2048
NCHUNK = 16
ROWS = CHUNK // 128
NGRID = 4
STRIPE = NBINS // 16


def _splat_body(pos_hbm, sx_hbm, sy_hbm, out_hbm,
                pxb, pyb, sxb, syb,
                pxb2, pyb2, sxb2, syb2,
                i00b, i10b, i01b, i11b,
                v00b, v10b, v01b, v11b,
                tmp, grid_sh, sem):
    cid = lax.axis_index("c")
    sid = lax.axis_index("s")
    goff = (sid // 4) * NBINS
    iota16 = lax.iota(jnp.int32, 16)

    @pl.loop(0, STRIPE // 16)
    def _(i):
        tmp[pl.ds(i * 16, 16)] = jnp.zeros((16,), jnp.float32)

    for g in range(NGRID):
        pltpu.sync_copy(tmp, grid_sh.at[pl.ds(g * NBINS + sid * STRIPE,
                                              STRIPE)])
    plsc.subcore_barrier()

    wid = cid * 16 + sid

    def bases(t):
        raw = (t * NW + wid) * CHUNK
        return jnp.minimum(raw, N_NODES - CHUNK), raw

    def load_descs(t, bufs):
        pxl, pyl, sxl, syl = bufs
        base, _ = bases(t)
        return [(pos_hbm.at[pl.ds(base, CHUNK)], pxl),
                (pos_hbm.at[pl.ds(N_NODES + base, CHUNK)], pyl),
                (sx_hbm.at[pl.ds(base, CHUNK)], sxl),
                (sy_hbm.at[pl.ds(base, CHUNK)], syl)]

    def fire_loads(t, bufs):
        for src, dst in load_descs(t, bufs):
            pltpu.async_copy(src, dst, sem)

    def wait_loads(t, bufs):
        for src, dst in load_descs(t, bufs):
            pltpu.make_async_copy(src, dst, sem).wait()

    def process(t, bufs):
        pxl, pyl, sxl, syl = bufs
        base, raw = bases(t)

        @pl.when(raw < N_NODES)
        def _():
            @pl.loop(0, ROWS)
            def _(r):
                def corners(sl, valid):
                    gx = pxl[sl] * INV_BSX - 0.5
                    gy = pyl[sl] * INV_BSY - 0.5
                    ix0 = gx.astype(jnp.int32)
                    iy0 = gy.astype(jnp.int32)
                    wx = jnp.clip(gx - ix0.astype(jnp.float32), 0.0, 1.0)
                    wy = jnp.clip(gy - iy0.astype(jnp.float32), 0.0, 1.0)
                    area = jnp.where(valid, sxl[sl] * syl[sl], 0.0)
                    iy0g = jnp.minimum(iy0, NBY - 1) + goff
                    iy1g = jnp.minimum(iy0 + 1, NBY - 1) + goff
                    bx0 = ix0 * NBY
                    bx1 = jnp.minimum(ix0 + 1, NBX - 1) * NBY
                    ax1 = wx * area
                    ax0 = area - ax1
                    return ((bx0 + iy0g, bx1 + iy0g, bx0 + iy1g,
                             bx1 + iy1g),
                            (ax0 - ax0 * wy, ax1 - ax1 * wy, ax0 * wy,
                             ax1 * wy))

                for c in range(8):
                    off = r * 128 + c * 16
                    sl = pl.ds(c * 16, 16)
                    idxs, vals = corners(pl.ds(off, 16),
                                         base + off + iota16 >= raw)
                    for ibuf, x in zip((i00b, i10b, i01b, i11b), idxs):
                        ibuf[sl] = x
                    for vbuf, x in zip((v00b, v10b, v01b, v11b), vals):
                        vbuf[sl] = x
                pltpu.sync_copy(v00b, grid_sh.at[i00b], add=True)
                pltpu.sync_copy(v10b, grid_sh.at[i10b], add=True)
                pltpu.sync_copy(v01b, grid_sh.at[i01b], add=True)
                pltpu.sync_copy(v11b, grid_sh.at[i11b], add=True)

    bufs_a = (pxb, pyb, sxb, syb)
    bufs_b = (pxb2, pyb2, sxb2, syb2)
    fire_loads(0, bufs_a)

    @pl.loop(0, NCHUNK // 2)
    def _(i):
        t0 = i * 2
        wait_loads(t0, bufs_a)
        fire_loads(t0 + 1, bufs_b)
        process(t0, bufs_a)
        wait_loads(t0 + 1, bufs_b)

        @pl.when(t0 + 2 < NCHUNK)
        def _():
            fire_loads(t0 + 2, bufs_a)

        process(t0 + 1, bufs_b)

    plsc.subcore_barrier()

    for g in range(NGRID):
        src = pl.ds(g * NBINS + sid * STRIPE, STRIPE)
        dst = pl.ds((cid * NGRID + g) * NBINS + sid * STRIPE, STRIPE)
        pltpu.sync_copy(grid_sh.at[src], tmp)
        pltpu.sync_copy(tmp, out_hbm.at[dst])


def _splat(pos, sx, sy):
    mesh = plsc.VectorSubcoreMesh(core_axis_name="c", subcore_axis_name="s")
    cp = pltpu.CompilerParams()
    if "needs_layout_passes" in pltpu.CompilerParams.__dataclass_fields__:
        cp = dataclasses.replace(cp, needs_layout_passes=False)
    k = pl.kernel(
        _splat_body,
        compiler_params=cp,
        out_type=jax.ShapeDtypeStruct((2 * NGRID * NBINS,), jnp.float32),
        mesh=mesh,
        scratch_types=(
            [pltpu.VMEM((CHUNK,), jnp.float32)] * 8
            + [pltpu.VMEM((128,), jnp.int32)] * 4
            + [pltpu.VMEM((128,), jnp.float32)] * 4
            + [
                pltpu.VMEM((STRIPE,), jnp.float32),
                pltpu.VMEM_SHARED((NGRID * NBINS,), jnp.float32),
                pltpu.SemaphoreType.DMA,
            ]
        ),
    )
    return k(pos, sx, sy)


def _dense_body(p_ref, out_ref):
    acc = p_ref[0].astype(jnp.float32)
    for g in range(1, 2 * NGRID):
        acc = acc + p_ref[g].astype(jnp.float32)
    rho = acc * (1.0 / BIN_AREA)
    rhs = rho - jnp.mean(rho)
    src = (0.25 * H2) * rhs

    def step(_, phi):
        up = jnp.concatenate([phi[:1, :], phi[:-1, :]], axis=0)
        down = jnp.concatenate([phi[1:, :], phi[-1:, :]], axis=0)
        left = jnp.concatenate([phi[:, :1], phi[:, :-1]], axis=1)
        right = jnp.concatenate([phi[:, 1:], phi[:, -1:]], axis=1)
        return 0.25 * (up + down + left + right) - src

    phi = lax.fori_loop(0, N_ITERS, step, jnp.zeros((NBX, NBY), jnp.float32))
    vx = jnp.concatenate([
        -(phi[1:2, :] - phi[0:1, :]) * INV_BSX,
        -(phi[2:, :] - phi[:-2, :]) * (0.5 * INV_BSX),
        -(phi[-1:, :] - phi[-2:-1, :]) * INV_BSX,
    ], axis=0)
    vy = jnp.concatenate([
        -(phi[:, 1:2] - phi[:, 0:1]) * INV_BSY,
        -(phi[:, 2:] - phi[:, :-2]) * (0.5 * INV_BSY),
        -(phi[:, -1:] - phi[:, -2:-1]) * INV_BSY,
    ], axis=1)
    energy = 0.5 * jnp.sum(rho * (vx * vx + vy * vy)) * BIN_AREA
    out_ref[...] = jnp.broadcast_to(energy, (1, 1))


def _dense(partials):
    return pl.pallas_call(
        _dense_body,
        out_shape=jax.ShapeDtypeStruct((1, 1), jnp.float32),
    )(partials)


def kernel(pos, node_size_x, node_size_y):
    flat = _splat(pos, node_size_x, node_size_y)
    partials = flat.reshape(2 * NGRID, NBX, NBY)
    energy = _dense(partials)
    return energy.reshape(1)


# speedup vs baseline: 1.2702x; 1.0707x over previous
import dataclasses

import jax
import jax.numpy as jnp
from jax import lax
from jax.experimental import pallas as pl
from jax.experimental.pallas import tpu as pltpu
from jax.experimental.pallas import tpu_sc as plsc

NBX = 512
NBY = 512
NBINS = NBX * NBY
BSX = 1.0 / NBX
BSY = 1.0 / NBY
INV_BSX = float(NBX)
INV_BSY = float(NBY)
BIN_AREA = BSX * BSY
H2 = BSX * BSY
N_ITERS = 40

N_NODES = 1_000_000
NW = 32
CHUNK = 2048
NCHUNK = 16
ROWS = CHUNK // 128
NGRID = 1
STRIPE = NBINS // 16


def _splat_body(pos_hbm, sx_hbm, sy_hbm, out_hbm,
                pxb, pyb, sxb, syb,
                pxb2, pyb2, sxb2, syb2,
                i00b, i10b, i01b, i11b,
                v00b, v10b, v01b, v11b,
                tmp, grid_sh, sem):
    cid = lax.axis_index("c")
    sid = lax.axis_index("s")
    iota16 = lax.iota(jnp.int32, 16)

    @pl.loop(0, STRIPE // 16)
    def _(i):
        tmp[pl.ds(i * 16, 16)] = jnp.zeros((16,), jnp.float32)

    for g in range(NGRID):
        pltpu.sync_copy(tmp, grid_sh.at[pl.ds(g * NBINS + sid * STRIPE,
                                              STRIPE)])
    plsc.subcore_barrier()

    wid = cid * 16 + sid

    def bases(t):
        raw = (t * NW + wid) * CHUNK
        return jnp.minimum(raw, N_NODES - CHUNK), raw

    def load_descs(t, bufs):
        pxl, pyl, sxl, syl = bufs
        base, _ = bases(t)
        return [(pos_hbm.at[pl.ds(base, CHUNK)], pxl),
                (pos_hbm.at[pl.ds(N_NODES + base, CHUNK)], pyl),
                (sx_hbm.at[pl.ds(base, CHUNK)], sxl),
                (sy_hbm.at[pl.ds(base, CHUNK)], syl)]

    def fire_loads(t, bufs):
        for src, dst in load_descs(t, bufs):
            pltpu.async_copy(src, dst, sem)

    def wait_loads(t, bufs):
        for src, dst in load_descs(t, bufs):
            pltpu.make_async_copy(src, dst, sem).wait()

    def process(t, bufs):
        pxl, pyl, sxl, syl = bufs
        base, raw = bases(t)

        @pl.when(raw < N_NODES)
        def _():
            @pl.loop(0, ROWS)
            def _(r):
                def corners(sl, valid):
                    gx = pxl[sl] * INV_BSX - 0.5
                    gy = pyl[sl] * INV_BSY - 0.5
                    ix0 = gx.astype(jnp.int32)
                    iy0 = gy.astype(jnp.int32)
                    wx = jnp.clip(gx - ix0.astype(jnp.float32), 0.0, 1.0)
                    wy = jnp.clip(gy - iy0.astype(jnp.float32), 0.0, 1.0)
                    area = jnp.where(valid, sxl[sl] * syl[sl], 0.0)
                    iy0g = jnp.minimum(iy0, NBY - 1)
                    iy1g = jnp.minimum(iy0 + 1, NBY - 1)
                    bx0 = ix0 * NBY
                    bx1 = jnp.minimum(ix0 + 1, NBX - 1) * NBY
                    ax1 = wx * area
                    ax0 = area - ax1
                    return ((bx0 + iy0g, bx1 + iy0g, bx0 + iy1g,
                             bx1 + iy1g),
                            (ax0 - ax0 * wy, ax1 - ax1 * wy, ax0 * wy,
                             ax1 * wy))

                for c in range(8):
                    off = r * 128 + c * 16
                    sl = pl.ds(c * 16, 16)
                    idxs, vals = corners(pl.ds(off, 16),
                                         base + off + iota16 >= raw)
                    for ibuf, x in zip((i00b, i10b, i01b, i11b), idxs):
                        ibuf[sl] = x
                    for vbuf, x in zip((v00b, v10b, v01b, v11b), vals):
                        vbuf[sl] = x
                pltpu.sync_copy(v00b, grid_sh.at[i00b], add=True)
                pltpu.sync_copy(v10b, grid_sh.at[i10b], add=True)
                pltpu.sync_copy(v01b, grid_sh.at[i01b], add=True)
                pltpu.sync_copy(v11b, grid_sh.at[i11b], add=True)

    bufs_a = (pxb, pyb, sxb, syb)
    bufs_b = (pxb2, pyb2, sxb2, syb2)
    fire_loads(0, bufs_a)

    @pl.loop(0, NCHUNK // 2)
    def _(i):
        t0 = i * 2
        wait_loads(t0, bufs_a)
        fire_loads(t0 + 1, bufs_b)
        process(t0, bufs_a)
        wait_loads(t0 + 1, bufs_b)

        @pl.when(t0 + 2 < NCHUNK)
        def _():
            fire_loads(t0 + 2, bufs_a)

        process(t0 + 1, bufs_b)

    plsc.subcore_barrier()

    for g in range(NGRID):
        src = pl.ds(g * NBINS + sid * STRIPE, STRIPE)
        dst = pl.ds((cid * NGRID + g) * NBINS + sid * STRIPE, STRIPE)
        pltpu.sync_copy(grid_sh.at[src], tmp)
        pltpu.sync_copy(tmp, out_hbm.at[dst])


def _splat(pos, sx, sy):
    mesh = plsc.VectorSubcoreMesh(core_axis_name="c", subcore_axis_name="s")
    cp = pltpu.CompilerParams()
    if "needs_layout_passes" in pltpu.CompilerParams.__dataclass_fields__:
        cp = dataclasses.replace(cp, needs_layout_passes=False)
    k = pl.kernel(
        _splat_body,
        compiler_params=cp,
        out_type=jax.ShapeDtypeStruct((2 * NGRID * NBINS,), jnp.float32),
        mesh=mesh,
        scratch_types=(
            [pltpu.VMEM((CHUNK,), jnp.float32)] * 8
            + [pltpu.VMEM((128,), jnp.int32)] * 4
            + [pltpu.VMEM((128,), jnp.float32)] * 4
            + [
                pltpu.VMEM((STRIPE,), jnp.float32),
                pltpu.VMEM_SHARED((NGRID * NBINS,), jnp.float32),
                pltpu.SemaphoreType.DMA,
            ]
        ),
    )
    return k(pos, sx, sy)


def _dense_body(p_ref, out_ref):
    acc = p_ref[0].astype(jnp.float32)
    for g in range(1, 2 * NGRID):
        acc = acc + p_ref[g].astype(jnp.float32)
    rho = acc * (1.0 / BIN_AREA)
    rhs = rho - jnp.mean(rho)
    src = (0.25 * H2) * rhs

    def step(_, phi):
        up = jnp.concatenate([phi[:1, :], phi[:-1, :]], axis=0)
        down = jnp.concatenate([phi[1:, :], phi[-1:, :]], axis=0)
        left = jnp.concatenate([phi[:, :1], phi[:, :-1]], axis=1)
        right = jnp.concatenate([phi[:, 1:], phi[:, -1:]], axis=1)
        return 0.25 * (up + down + left + right) - src

    phi = lax.fori_loop(0, N_ITERS, step, jnp.zeros((NBX, NBY), jnp.float32))
    vx = jnp.concatenate([
        -(phi[1:2, :] - phi[0:1, :]) * INV_BSX,
        -(phi[2:, :] - phi[:-2, :]) * (0.5 * INV_BSX),
        -(phi[-1:, :] - phi[-2:-1, :]) * INV_BSX,
    ], axis=0)
    vy = jnp.concatenate([
        -(phi[:, 1:2] - phi[:, 0:1]) * INV_BSY,
        -(phi[:, 2:] - phi[:, :-2]) * (0.5 * INV_BSY),
        -(phi[:, -1:] - phi[:, -2:-1]) * INV_BSY,
    ], axis=1)
    energy = 0.5 * jnp.sum(rho * (vx * vx + vy * vy)) * BIN_AREA
    out_ref[...] = jnp.broadcast_to(energy, (1, 1))


def _dense(partials):
    return pl.pallas_call(
        _dense_body,
        out_shape=jax.ShapeDtypeStruct((1, 1), jnp.float32),
    )(partials)


def kernel(pos, node_size_x, node_size_y):
    flat = _splat(pos, node_size_x, node_size_y)
    partials = flat.reshape(2 * NGRID, NBX, NBY)
    energy = _dense(partials)
    return energy.reshape(1)


# speedup vs baseline: 1.6788x; 1.3217x over previous
import dataclasses

import jax
import jax.numpy as jnp
from jax import lax
from jax.experimental import pallas as pl
from jax.experimental.pallas import tpu as pltpu
from jax.experimental.pallas import tpu_sc as plsc

NBX = 512
NBY = 512
NBINS = NBX * NBY
BSX = 1.0 / NBX
BSY = 1.0 / NBY
INV_BSX = float(NBX)
INV_BSY = float(NBY)
BIN_AREA = BSX * BSY
H2 = BSX * BSY
N_ITERS = 40

N_NODES = 1_000_000
NW = 32
CHUNK = 2048
NCHUNK = 16
ROWS = CHUNK // 128
NGRID = 1
STRIPE = NBINS // 16


def _splat_body(pos_hbm, sx_hbm, sy_hbm, out_hbm,
                pxb, pyb, sxb, syb,
                pxb2, pyb2, sxb2, syb2,
                i00b, i10b, i01b, i11b,
                v00b, v10b, v01b, v11b,
                tmp, grid_sh, sem, ssem):
    cid = lax.axis_index("c")
    sid = lax.axis_index("s")
    iota16 = lax.iota(jnp.int32, 16)

    @pl.loop(0, STRIPE // 16)
    def _(i):
        tmp[pl.ds(i * 16, 16)] = jnp.zeros((16,), jnp.float32)

    for g in range(NGRID):
        pltpu.sync_copy(tmp, grid_sh.at[pl.ds(g * NBINS + sid * STRIPE,
                                              STRIPE)])
    plsc.subcore_barrier()

    wid = cid * 16 + sid

    def bases(t):
        raw = (t * NW + wid) * CHUNK
        return jnp.minimum(raw, N_NODES - CHUNK), raw

    def load_descs(t, bufs):
        pxl, pyl, sxl, syl = bufs
        base, _ = bases(t)
        return [(pos_hbm.at[pl.ds(base, CHUNK)], pxl),
                (pos_hbm.at[pl.ds(N_NODES + base, CHUNK)], pyl),
                (sx_hbm.at[pl.ds(base, CHUNK)], sxl),
                (sy_hbm.at[pl.ds(base, CHUNK)], syl)]

    def fire_loads(t, bufs):
        for src, dst in load_descs(t, bufs):
            pltpu.async_copy(src, dst, sem)

    def wait_loads(t, bufs):
        for src, dst in load_descs(t, bufs):
            pltpu.make_async_copy(src, dst, sem).wait()

    def process(t, bufs):
        pxl, pyl, sxl, syl = bufs
        base, raw = bases(t)

        @pl.when(raw < N_NODES)
        def _():
            @pl.loop(0, ROWS)
            def _(r):
                def corners(sl, valid):
                    gx = pxl[sl] * INV_BSX - 0.5
                    gy = pyl[sl] * INV_BSY - 0.5
                    ix0 = gx.astype(jnp.int32)
                    iy0 = gy.astype(jnp.int32)
                    wx = jnp.clip(gx - ix0.astype(jnp.float32), 0.0, 1.0)
                    wy = jnp.clip(gy - iy0.astype(jnp.float32), 0.0, 1.0)
                    area = jnp.where(valid, sxl[sl] * syl[sl], 0.0)
                    iy0g = jnp.minimum(iy0, NBY - 1)
                    iy1g = jnp.minimum(iy0 + 1, NBY - 1)
                    bx0 = ix0 * NBY
                    bx1 = jnp.minimum(ix0 + 1, NBX - 1) * NBY
                    ax1 = wx * area
                    ax0 = area - ax1
                    return ((bx0 + iy0g, bx1 + iy0g, bx0 + iy1g,
                             bx1 + iy1g),
                            (ax0 - ax0 * wy, ax1 - ax1 * wy, ax0 * wy,
                             ax1 * wy))

                for c in range(8):
                    off = r * 128 + c * 16
                    sl = pl.ds(c * 16, 16)
                    idxs, vals = corners(pl.ds(off, 16),
                                         base + off + iota16 >= raw)
                    for ibuf, x in zip((i00b, i10b, i01b, i11b), idxs):
                        ibuf[sl] = x
                    for vbuf, x in zip((v00b, v10b, v01b, v11b), vals):
                        vbuf[sl] = x
                cs = [
                    pltpu.async_copy(v00b, grid_sh.at[i00b], ssem, add=True),
                    pltpu.async_copy(v10b, grid_sh.at[i10b], ssem, add=True),
                    pltpu.async_copy(v01b, grid_sh.at[i01b], ssem, add=True),
                    pltpu.async_copy(v11b, grid_sh.at[i11b], ssem, add=True),
                ]
                for c_ in cs:
                    c_.wait()

    bufs_a = (pxb, pyb, sxb, syb)
    bufs_b = (pxb2, pyb2, sxb2, syb2)
    fire_loads(0, bufs_a)

    @pl.loop(0, NCHUNK // 2)
    def _(i):
        t0 = i * 2
        wait_loads(t0, bufs_a)
        fire_loads(t0 + 1, bufs_b)
        process(t0, bufs_a)
        wait_loads(t0 + 1, bufs_b)

        @pl.when(t0 + 2 < NCHUNK)
        def _():
            fire_loads(t0 + 2, bufs_a)

        process(t0 + 1, bufs_b)

    plsc.subcore_barrier()

    for g in range(NGRID):
        src = pl.ds(g * NBINS + sid * STRIPE, STRIPE)
        dst = pl.ds((cid * NGRID + g) * NBINS + sid * STRIPE, STRIPE)
        pltpu.sync_copy(grid_sh.at[src], tmp)
        pltpu.sync_copy(tmp, out_hbm.at[dst])


def _splat(pos, sx, sy):
    mesh = plsc.VectorSubcoreMesh(core_axis_name="c", subcore_axis_name="s")
    cp = pltpu.CompilerParams()
    if "needs_layout_passes" in pltpu.CompilerParams.__dataclass_fields__:
        cp = dataclasses.replace(cp, needs_layout_passes=False)
    k = pl.kernel(
        _splat_body,
        compiler_params=cp,
        out_type=jax.ShapeDtypeStruct((2 * NGRID * NBINS,), jnp.float32),
        mesh=mesh,
        scratch_types=(
            [pltpu.VMEM((CHUNK,), jnp.float32)] * 8
            + [pltpu.VMEM((128,), jnp.int32)] * 4
            + [pltpu.VMEM((128,), jnp.float32)] * 4
            + [
                pltpu.VMEM((STRIPE,), jnp.float32),
                pltpu.VMEM_SHARED((NGRID * NBINS,), jnp.float32),
                pltpu.SemaphoreType.DMA,
                pltpu.SemaphoreType.DMA,
            ]
        ),
    )
    return k(pos, sx, sy)


def _dense_body(p_ref, out_ref):
    acc = p_ref[0].astype(jnp.float32)
    for g in range(1, 2 * NGRID):
        acc = acc + p_ref[g].astype(jnp.float32)
    rho = acc * (1.0 / BIN_AREA)
    rhs = rho - jnp.mean(rho)
    src = (0.25 * H2) * rhs

    def step(_, phi):
        up = jnp.concatenate([phi[:1, :], phi[:-1, :]], axis=0)
        down = jnp.concatenate([phi[1:, :], phi[-1:, :]], axis=0)
        left = jnp.concatenate([phi[:, :1], phi[:, :-1]], axis=1)
        right = jnp.concatenate([phi[:, 1:], phi[:, -1:]], axis=1)
        return 0.25 * (up + down + left + right) - src

    phi = lax.fori_loop(0, N_ITERS, step, jnp.zeros((NBX, NBY), jnp.float32))
    vx = jnp.concatenate([
        -(phi[1:2, :] - phi[0:1, :]) * INV_BSX,
        -(phi[2:, :] - phi[:-2, :]) * (0.5 * INV_BSX),
        -(phi[-1:, :] - phi[-2:-1, :]) * INV_BSX,
    ], axis=0)
    vy = jnp.concatenate([
        -(phi[:, 1:2] - phi[:, 0:1]) * INV_BSY,
        -(phi[:, 2:] - phi[:, :-2]) * (0.5 * INV_BSY),
        -(phi[:, -1:] - phi[:, -2:-1]) * INV_BSY,
    ], axis=1)
    energy = 0.5 * jnp.sum(rho * (vx * vx + vy * vy)) * BIN_AREA
    out_ref[...] = jnp.broadcast_to(energy, (1, 1))


def _dense(partials):
    return pl.pallas_call(
        _dense_body,
        out_shape=jax.ShapeDtypeStruct((1, 1), jnp.float32),
    )(partials)


def kernel(pos, node_size_x, node_size_y):
    flat = _splat(pos, node_size_x, node_size_y)
    partials = flat.reshape(2 * NGRID, NBX, NBY)
    energy = _dense(partials)
    return energy.reshape(1)
